# Initial kernel scaffold; baseline (speedup 1.0000x reference)
import jax
import jax.numpy as jnp
from jax.experimental import pallas as pl


def kernel(x, edge_index, batch, Wg_sep, Wg_enc, Wsep1, bsep1, Wsep2, bsep2, codebook, Wc1, bc1, Wc2, bc2, bn_gamma, bn_beta):
    raise NotImplementedError("write your pallas kernel here")


# speedup vs baseline: 6.9051x; 6.9051x over previous
import functools

import jax
import jax.numpy as jnp
from jax import lax
from jax.experimental import pallas as pl
from jax.experimental.pallas import tpu as pltpu
from jax.experimental.pallas import tpu_sc as plsc

N = 10000
E = 320000
D = 128
G = 512
K = 1024
COMMIT = 0.25
GAMMA = 0.5

NC = 2
NS = 16
NW = NC * NS
E_PER_TILE = E // NW
CHUNK = 80
NCHUNK = E_PER_TILE // CHUNK
NPAD = 10240
ROWS_PER_TILE = NPAD // NS


def _edge_agg_kernel(x_hbm, src_hbm, dst_hbm, zeros_hbm, out_hbm,
                     src_i, dst_i, rows_v, acc_sh, sem):
    cid = lax.axis_index("c")
    sid = lax.axis_index("s")
    wid = cid * NS + sid
    pltpu.sync_copy(zeros_hbm.at[pl.ds(sid * ROWS_PER_TILE, ROWS_PER_TILE)],
                    acc_sh.at[pl.ds(sid * ROWS_PER_TILE, ROWS_PER_TILE)])
    plsc.subcore_barrier()
    pltpu.sync_copy(src_hbm.at[wid], src_i)
    pltpu.sync_copy(dst_hbm.at[wid], dst_i)

    def body(j, carry):
        pltpu.async_copy(x_hbm.at[src_i.at[j]], rows_v, sem).wait()
        pltpu.sync_copy(rows_v, acc_sh.at[dst_i.at[j]], add=True)
        return carry

    lax.fori_loop(0, NCHUNK, body, 0)
    plsc.subcore_barrier()
    pltpu.sync_copy(acc_sh.at[pl.ds(sid * ROWS_PER_TILE, ROWS_PER_TILE)],
                    out_hbm.at[cid, pl.ds(sid * ROWS_PER_TILE, ROWS_PER_TILE)])


@functools.cache
def _edge_agg():
    return pl.kernel(
        _edge_agg_kernel,
        mesh=plsc.VectorSubcoreMesh(core_axis_name="c", subcore_axis_name="s"),
        out_type=jax.ShapeDtypeStruct((NC, NPAD, D), jnp.float32),
        scratch_types=[
            pltpu.VMEM((NCHUNK, CHUNK), jnp.int32),
            pltpu.VMEM((NCHUNK, CHUNK), jnp.int32),
            pltpu.VMEM((CHUNK, D), jnp.float32),
            pltpu.VMEM_SHARED((NPAD, D), jnp.float32),
            pltpu.SemaphoreType.DMA,
        ],
    )


N_BLK = 1000
N_GRID = N // N_BLK


def _node_kernel(x_ref, p0_ref, p1_ref, b_ref,
                 wgs_ref, wge_ref, ws1_ref, bs1_ref, ws2_ref, bs2_ref,
                 cb_ref,
                 c_sum_ref, s_sum_ref, aux_ref):
    i = pl.program_id(0)
    f32 = jnp.float32
    y = x_ref[...] + p0_ref[...] + p1_ref[...]
    h_sep = jnp.maximum(jnp.dot(y, wgs_ref[...], preferred_element_type=f32), 0.0)
    s_hid = jnp.maximum(jnp.dot(h_sep, ws1_ref[...], preferred_element_type=f32)
                        + bs1_ref[...], 0.0)
    score = jax.nn.sigmoid(jnp.dot(s_hid, ws2_ref[...], preferred_element_type=f32)
                           + bs2_ref[...])
    pos = jnp.mean(score, axis=1, keepdims=True)
    nf = jnp.maximum(jnp.dot(y, wge_ref[...], preferred_element_type=f32), 0.0)
    cb = cb_ref[...]
    d2 = (jnp.sum(nf * nf, axis=1, keepdims=True)
          - 2.0 * jnp.dot(nf, cb.T, preferred_element_type=f32)
          + jnp.sum(cb * cb, axis=1)[None, :])
    m = jnp.min(d2, axis=1, keepdims=True)
    kio = lax.broadcasted_iota(jnp.int32, (N_BLK, K), 1)
    idx = jnp.min(jnp.where(d2 == m, kio, K), axis=1)
    onehot = (kio == idx[:, None]).astype(f32)
    quant = jnp.dot(onehot, cb, preferred_element_type=f32)
    res = nf + quant
    c_node = res * score
    s_node = res * (1.0 - score)
    cmt_row = jnp.sum((nf - quant) ** 2, axis=1, keepdims=True)
    b = b_ref[0]
    giota = lax.broadcasted_iota(jnp.int32, (G, N_BLK), 0)
    seg = (giota == b).astype(f32)
    ones = jnp.ones((N_BLK, 1), f32)
    zeros4 = jnp.zeros((N_BLK, 4), f32)
    auxv = jnp.concatenate([pos, 1.0 - pos, ones, cmt_row, zeros4], axis=1)
    c_part = jnp.dot(seg, c_node, preferred_element_type=f32)
    s_part = jnp.dot(seg, s_node, preferred_element_type=f32)
    a_part = jnp.dot(seg, auxv, preferred_element_type=f32)

    @pl.when(i == 0)
    def _():
        c_sum_ref[...] = c_part
        s_sum_ref[...] = s_part
        aux_ref[...] = a_part

    @pl.when(i > 0)
    def _():
        c_sum_ref[...] += c_part
        s_sum_ref[...] += s_part
        aux_ref[...] += a_part


def _head_kernel(c_sum_ref, s_sum_ref, aux_ref,
                 wc1_ref, bc1_ref, wc2_ref, bc2_ref, gam_ref, bet_ref,
                 logit_ref, cg_ref, sg_ref, scal_ref):
    f32 = jnp.float32
    aux = aux_ref[...]
    cnt = jnp.maximum(aux[:, 2:3], 1.0)
    c_graph = c_sum_ref[...] / cnt
    s_graph = s_sum_ref[...] / cnt
    cg_ref[...] = c_graph
    sg_ref[...] = s_graph
    h = jnp.dot(c_graph, wc1_ref[...], preferred_element_type=f32) + bc1_ref[...]
    mu = jnp.mean(h, axis=0, keepdims=True)
    var = jnp.mean((h - mu) ** 2, axis=0, keepdims=True)
    h = (h - mu) / jnp.sqrt(var + 1e-5) * gam_ref[...] + bet_ref[...]
    h = jnp.maximum(h, 0.0)
    logit_ref[...] = jnp.dot(h, wc2_ref[...], preferred_element_type=f32) + bc2_ref[...]
    pos_s = aux[:, 0:1] + 1e-8
    neg_s = aux[:, 1:2] + 1e-8
    loss_reg = jnp.mean(jnp.abs(pos_s / (pos_s + neg_s) - GAMMA))
    cmt_loss = COMMIT * jnp.sum(aux[:, 3:4]) / (N * D)
    scal_ref[...] = jnp.concatenate(
        [jnp.full((1, 1), cmt_loss, f32), jnp.full((1, 1), loss_reg, f32)], axis=1)


def kernel(x, edge_index, batch, Wg_sep, Wg_enc, Wsep1, bsep1, Wsep2, bsep2,
           codebook, Wc1, bc1, Wc2, bc2, bn_gamma, bn_beta):
    src = edge_index[0].reshape(NW, NCHUNK, CHUNK)
    dst = edge_index[1].reshape(NW, NCHUNK, CHUNK)
    zeros = jnp.zeros((NPAD, D), jnp.float32)
    parts = _edge_agg()(x, src, dst, zeros)
    parts = parts[:, :N, :]

    batch3 = batch.reshape(N_GRID, 1, N_BLK)
    full = lambda *s: pl.BlockSpec(s, lambda i: (0,) * len(s))
    c_sum, s_sum, aux = pl.pallas_call(
        _node_kernel,
        grid=(N_GRID,),
        in_specs=[
            pl.BlockSpec((N_BLK, D), lambda i: (i, 0)),
            pl.BlockSpec((N_BLK, D), lambda i: (i, 0)),
            pl.BlockSpec((N_BLK, D), lambda i: (i, 0)),
            pl.BlockSpec((1, 1, N_BLK), lambda i: (i, 0, 0)),
            full(D, D), full(D, D), full(D, 2 * D), full(1, 2 * D),
            full(2 * D, D), full(1, D), full(K, D),
        ],
        out_specs=[
            pl.BlockSpec((G, D), lambda i: (0, 0)),
            pl.BlockSpec((G, D), lambda i: (0, 0)),
            pl.BlockSpec((G, 8), lambda i: (0, 0)),
        ],
        out_shape=[
            jax.ShapeDtypeStruct((G, D), jnp.float32),
            jax.ShapeDtypeStruct((G, D), jnp.float32),
            jax.ShapeDtypeStruct((G, 8), jnp.float32),
        ],
    )(x, parts[0], parts[1], batch3,
      Wg_sep, Wg_enc, Wsep1, bsep1.reshape(1, 2 * D), Wsep2,
      bsep2.reshape(1, D), codebook)

    c_logit, c_graph, s_graph, scal = pl.pallas_call(
        _head_kernel,
        out_shape=[
            jax.ShapeDtypeStruct((G, 1), jnp.float32),
            jax.ShapeDtypeStruct((G, D), jnp.float32),
            jax.ShapeDtypeStruct((G, D), jnp.float32),
            jax.ShapeDtypeStruct((1, 2), jnp.float32),
        ],
    )(c_sum, s_sum, aux, Wc1, bc1.reshape(1, 2 * D), Wc2, bc2.reshape(1, 1),
      bn_gamma.reshape(1, 2 * D), bn_beta.reshape(1, 2 * D))

    return (c_logit, c_graph, s_graph, scal[0, 0], scal[0, 1])


# speedup vs baseline: 10.0483x; 1.4552x over previous
import functools

import jax
import jax.numpy as jnp
from jax import lax
from jax.experimental import pallas as pl
from jax.experimental.pallas import tpu as pltpu
from jax.experimental.pallas import tpu_sc as plsc

N = 10000
E = 320000
D = 128
G = 512
K = 1024
COMMIT = 0.25
GAMMA = 0.5

NC = 2
NS = 16
NW = NC * NS
E_PER_TILE = E // NW
CHUNK = 40
NCHUNK = E_PER_TILE // CHUNK
U = 10
NG = NCHUNK // U
R = 5
GLEAD = 2
ILEAD = 4
NPAD = 10240
ROWS_PER_TILE = NPAD // NS


def _edge_agg_kernel(x_hbm, src_hbm, dst_hbm, zeros_hbm, out_hbm,
                     srcb, dstb, r0, r1, r2, r3, r4, acc_sh,
                     isem, gsem, ssem):
    rows = [r0, r1, r2, r3, r4]
    cid = lax.axis_index("c")
    sid = lax.axis_index("s")
    wid = cid * NS + sid
    pltpu.sync_copy(zeros_hbm.at[pl.ds(sid * ROWS_PER_TILE, ROWS_PER_TILE)],
                    acc_sh.at[pl.ds(sid * ROWS_PER_TILE, ROWS_PER_TILE)])
    plsc.subcore_barrier()

    def fire_idx(c, s):
        pltpu.async_copy(src_hbm.at[wid, c], srcb.at[s], isem.at[s])
        pltpu.async_copy(dst_hbm.at[wid, c], dstb.at[s], isem.at[s])

    def wait_idx(s):
        pltpu.make_async_copy(src_hbm.at[wid, 0], srcb.at[s], isem.at[s]).wait()
        pltpu.make_async_copy(dst_hbm.at[wid, 0], dstb.at[s], isem.at[s]).wait()

    def fire_gather(s, r):
        pltpu.async_copy(x_hbm.at[srcb.at[s]], rows[r], gsem.at[r])

    def wait_gather(r):
        pltpu.make_async_copy(x_hbm.at[srcb.at[0]], rows[r], gsem.at[r]).wait()

    def fire_scatter(s, r):
        pltpu.async_copy(rows[r], acc_sh.at[dstb.at[s]], ssem.at[r], add=True)

    def wait_scatter(r):
        pltpu.make_async_copy(rows[r], acc_sh.at[dstb.at[0]],
                              ssem.at[r]).wait()

    for c in range(ILEAD):
        fire_idx(c, c)
    for c in range(GLEAD):
        wait_idx(c)
        fire_gather(c, c)

    def body(g, carry):
        for b in range(U):
            c = g * U + b
            ci = c + ILEAD
            si = (b + ILEAD) % U
            if b < U - ILEAD:
                fire_idx(ci, si)
            else:
                @pl.when(g <= NG - 2)
                def _():
                    fire_idx(ci, si)
            cg = c + GLEAD
            sg = (b + GLEAD) % U
            rg = (b + GLEAD) % R
            if b < U - GLEAD:
                if b < R - GLEAD:
                    @pl.when(g >= 1)
                    def _():
                        wait_scatter(rg)
                else:
                    wait_scatter(rg)
                wait_idx(sg)
                fire_gather(sg, rg)
            else:
                @pl.when(g <= NG - 2)
                def _():
                    wait_scatter(rg)
                    wait_idx(sg)
                    fire_gather(sg, rg)
            wait_gather(b % R)
            fire_scatter(b, b % R)
        return carry

    lax.fori_loop(0, NG, body, 0)
    for r in range(R):
        wait_scatter(r)
    plsc.subcore_barrier()
    pltpu.sync_copy(acc_sh.at[pl.ds(sid * ROWS_PER_TILE, ROWS_PER_TILE)],
                    out_hbm.at[cid, pl.ds(sid * ROWS_PER_TILE, ROWS_PER_TILE)])


@functools.cache
def _edge_agg():
    return pl.kernel(
        _edge_agg_kernel,
        mesh=plsc.VectorSubcoreMesh(core_axis_name="c", subcore_axis_name="s"),
        out_type=jax.ShapeDtypeStruct((NC, NPAD, D), jnp.float32),
        scratch_types=[
            pltpu.VMEM((U, CHUNK), jnp.int32),
            pltpu.VMEM((U, CHUNK), jnp.int32),
            pltpu.VMEM((CHUNK, D), jnp.float32),
            pltpu.VMEM((CHUNK, D), jnp.float32),
            pltpu.VMEM((CHUNK, D), jnp.float32),
            pltpu.VMEM((CHUNK, D), jnp.float32),
            pltpu.VMEM((CHUNK, D), jnp.float32),
            pltpu.VMEM_SHARED((NPAD, D), jnp.float32),
            pltpu.SemaphoreType.DMA((U,)),
            pltpu.SemaphoreType.DMA((R,)),
            pltpu.SemaphoreType.DMA((R,)),
        ],
    )


N_BLK = 1000
N_GRID = N // N_BLK


def _node_kernel(x_ref, p0_ref, p1_ref, b_ref,
                 wgs_ref, wge_ref, ws1_ref, bs1_ref, ws2_ref, bs2_ref,
                 cb_ref,
                 c_sum_ref, s_sum_ref, aux_ref):
    i = pl.program_id(0)
    f32 = jnp.float32
    y = x_ref[...] + p0_ref[...] + p1_ref[...]
    h_sep = jnp.maximum(jnp.dot(y, wgs_ref[...], preferred_element_type=f32), 0.0)
    s_hid = jnp.maximum(jnp.dot(h_sep, ws1_ref[...], preferred_element_type=f32)
                        + bs1_ref[...], 0.0)
    score = jax.nn.sigmoid(jnp.dot(s_hid, ws2_ref[...], preferred_element_type=f32)
                           + bs2_ref[...])
    pos = jnp.mean(score, axis=1, keepdims=True)
    nf = jnp.maximum(jnp.dot(y, wge_ref[...], preferred_element_type=f32), 0.0)
    cb = cb_ref[...]
    d2 = (jnp.sum(nf * nf, axis=1, keepdims=True)
          - 2.0 * jnp.dot(nf, cb.T, preferred_element_type=f32)
          + jnp.sum(cb * cb, axis=1)[None, :])
    m = jnp.min(d2, axis=1, keepdims=True)
    kio = lax.broadcasted_iota(jnp.int32, (N_BLK, K), 1)
    idx = jnp.min(jnp.where(d2 == m, kio, K), axis=1)
    onehot = (kio == idx[:, None]).astype(f32)
    quant = jnp.dot(onehot, cb, preferred_element_type=f32)
    res = nf + quant
    c_node = res * score
    s_node = res * (1.0 - score)
    cmt_row = jnp.sum((nf - quant) ** 2, axis=1, keepdims=True)
    b = b_ref[0]
    giota = lax.broadcasted_iota(jnp.int32, (G, N_BLK), 0)
    seg = (giota == b).astype(f32)
    ones = jnp.ones((N_BLK, 1), f32)
    zeros4 = jnp.zeros((N_BLK, 4), f32)
    auxv = jnp.concatenate([pos, 1.0 - pos, ones, cmt_row, zeros4], axis=1)
    c_part = jnp.dot(seg, c_node, preferred_element_type=f32)
    s_part = jnp.dot(seg, s_node, preferred_element_type=f32)
    a_part = jnp.dot(seg, auxv, preferred_element_type=f32)

    @pl.when(i == 0)
    def _():
        c_sum_ref[...] = c_part
        s_sum_ref[...] = s_part
        aux_ref[...] = a_part

    @pl.when(i > 0)
    def _():
        c_sum_ref[...] += c_part
        s_sum_ref[...] += s_part
        aux_ref[...] += a_part


def _head_kernel(c_sum_ref, s_sum_ref, aux_ref,
                 wc1_ref, bc1_ref, wc2_ref, bc2_ref, gam_ref, bet_ref,
                 logit_ref, cg_ref, sg_ref, scal_ref):
    f32 = jnp.float32
    aux = aux_ref[...]
    cnt = jnp.maximum(aux[:, 2:3], 1.0)
    c_graph = c_sum_ref[...] / cnt
    s_graph = s_sum_ref[...] / cnt
    cg_ref[...] = c_graph
    sg_ref[...] = s_graph
    h = jnp.dot(c_graph, wc1_ref[...], preferred_element_type=f32) + bc1_ref[...]
    mu = jnp.mean(h, axis=0, keepdims=True)
    var = jnp.mean((h - mu) ** 2, axis=0, keepdims=True)
    h = (h - mu) / jnp.sqrt(var + 1e-5) * gam_ref[...] + bet_ref[...]
    h = jnp.maximum(h, 0.0)
    logit_ref[...] = jnp.dot(h, wc2_ref[...], preferred_element_type=f32) + bc2_ref[...]
    pos_s = aux[:, 0:1] + 1e-8
    neg_s = aux[:, 1:2] + 1e-8
    loss_reg = jnp.mean(jnp.abs(pos_s / (pos_s + neg_s) - GAMMA))
    cmt_loss = COMMIT * jnp.sum(aux[:, 3:4]) / (N * D)
    scal_ref[...] = jnp.concatenate(
        [jnp.full((1, 1), cmt_loss, f32), jnp.full((1, 1), loss_reg, f32)], axis=1)


def kernel(x, edge_index, batch, Wg_sep, Wg_enc, Wsep1, bsep1, Wsep2, bsep2,
           codebook, Wc1, bc1, Wc2, bc2, bn_gamma, bn_beta):
    src = edge_index[0].reshape(NW, NCHUNK, CHUNK)
    dst = edge_index[1].reshape(NW, NCHUNK, CHUNK)
    zeros = jnp.zeros((NPAD, D), jnp.float32)
    parts = _edge_agg()(x, src, dst, zeros)
    parts = parts[:, :N, :]

    batch3 = batch.reshape(N_GRID, 1, N_BLK)
    full = lambda *s: pl.BlockSpec(s, lambda i: (0,) * len(s))
    c_sum, s_sum, aux = pl.pallas_call(
        _node_kernel,
        grid=(N_GRID,),
        in_specs=[
            pl.BlockSpec((N_BLK, D), lambda i: (i, 0)),
            pl.BlockSpec((N_BLK, D), lambda i: (i, 0)),
            pl.BlockSpec((N_BLK, D), lambda i: (i, 0)),
            pl.BlockSpec((1, 1, N_BLK), lambda i: (i, 0, 0)),
            full(D, D), full(D, D), full(D, 2 * D), full(1, 2 * D),
            full(2 * D, D), full(1, D), full(K, D),
        ],
        out_specs=[
            pl.BlockSpec((G, D), lambda i: (0, 0)),
            pl.BlockSpec((G, D), lambda i: (0, 0)),
            pl.BlockSpec((G, 8), lambda i: (0, 0)),
        ],
        out_shape=[
            jax.ShapeDtypeStruct((G, D), jnp.float32),
            jax.ShapeDtypeStruct((G, D), jnp.float32),
            jax.ShapeDtypeStruct((G, 8), jnp.float32),
        ],
    )(x, parts[0], parts[1], batch3,
      Wg_sep, Wg_enc, Wsep1, bsep1.reshape(1, 2 * D), Wsep2,
      bsep2.reshape(1, D), codebook)

    c_logit, c_graph, s_graph, scal = pl.pallas_call(
        _head_kernel,
        out_shape=[
            jax.ShapeDtypeStruct((G, 1), jnp.float32),
            jax.ShapeDtypeStruct((G, D), jnp.float32),
            jax.ShapeDtypeStruct((G, D), jnp.float32),
            jax.ShapeDtypeStruct((1, 2), jnp.float32),
        ],
    )(c_sum, s_sum, aux, Wc1, bc1.reshape(1, 2 * D), Wc2, bc2.reshape(1, 1),
      bn_gamma.reshape(1, 2 * D), bn_beta.reshape(1, 2 * D))

    return (c_logit, c_graph, s_graph, scal[0, 0], scal[0, 1])


# speedup vs baseline: 10.5308x; 1.0480x over previous
import functools

import jax
import jax.numpy as jnp
from jax import lax
from jax.experimental import pallas as pl
from jax.experimental.pallas import tpu as pltpu
from jax.experimental.pallas import tpu_sc as plsc

N = 10000
E = 320000
D = 128
G = 512
K = 1024
COMMIT = 0.25
GAMMA = 0.5

NC = 2
NS = 16
NW = NC * NS
E_PER_TILE = E // NW
CHUNK = 40
NCHUNK = E_PER_TILE // CHUNK
U = 10
NG = NCHUNK // U
R = 5
GLEAD = 2
ILEAD = 4
NPAD = 10240
ROWS_PER_TILE = NPAD // NS


def _edge_agg_kernel(x_hbm, src_hbm, dst_hbm, zeros_hbm, out_hbm,
                     srcb, dstb, r0, r1, r2, r3, r4, acc_sh,
                     isem, gsem, ssem):
    rows = [r0, r1, r2, r3, r4]
    cid = lax.axis_index("c")
    sid = lax.axis_index("s")
    wid = cid * NS + sid
    pltpu.sync_copy(zeros_hbm.at[pl.ds(sid * ROWS_PER_TILE, ROWS_PER_TILE)],
                    acc_sh.at[pl.ds(sid * ROWS_PER_TILE, ROWS_PER_TILE)])
    plsc.subcore_barrier()

    base = wid * E_PER_TILE

    def fire_idx(c, s):
        off = base + c * CHUNK
        pltpu.async_copy(src_hbm.at[pl.ds(off, CHUNK)], srcb.at[s], isem.at[s])
        pltpu.async_copy(dst_hbm.at[pl.ds(off, CHUNK)], dstb.at[s], isem.at[s])

    def wait_idx(s):
        pltpu.make_async_copy(src_hbm.at[pl.ds(0, CHUNK)], srcb.at[s],
                              isem.at[s]).wait()
        pltpu.make_async_copy(dst_hbm.at[pl.ds(0, CHUNK)], dstb.at[s],
                              isem.at[s]).wait()

    def fire_gather(s, r):
        pltpu.async_copy(x_hbm.at[srcb.at[s]], rows[r], gsem.at[r])

    def wait_gather(r):
        pltpu.make_async_copy(x_hbm.at[srcb.at[0]], rows[r], gsem.at[r]).wait()

    def fire_scatter(s, r):
        pltpu.async_copy(rows[r], acc_sh.at[dstb.at[s]], ssem.at[r], add=True)

    def wait_scatter(r):
        pltpu.make_async_copy(rows[r], acc_sh.at[dstb.at[0]],
                              ssem.at[r]).wait()

    for c in range(ILEAD):
        fire_idx(c, c)
    for c in range(GLEAD):
        wait_idx(c)
        fire_gather(c, c)

    def body(g, carry):
        for b in range(U):
            c = g * U + b
            ci = c + ILEAD
            si = (b + ILEAD) % U
            if b < U - ILEAD:
                fire_idx(ci, si)
            else:
                @pl.when(g <= NG - 2)
                def _():
                    fire_idx(ci, si)
            cg = c + GLEAD
            sg = (b + GLEAD) % U
            rg = (b + GLEAD) % R
            if b < U - GLEAD:
                if b < R - GLEAD:
                    @pl.when(g >= 1)
                    def _():
                        wait_scatter(rg)
                else:
                    wait_scatter(rg)
                wait_idx(sg)
                fire_gather(sg, rg)
            else:
                @pl.when(g <= NG - 2)
                def _():
                    wait_scatter(rg)
                    wait_idx(sg)
                    fire_gather(sg, rg)
            wait_gather(b % R)
            fire_scatter(b, b % R)
        return carry

    lax.fori_loop(0, NG, body, 0)
    for r in range(R):
        wait_scatter(r)
    plsc.subcore_barrier()
    pltpu.sync_copy(acc_sh.at[pl.ds(sid * ROWS_PER_TILE, ROWS_PER_TILE)],
                    out_hbm.at[cid, pl.ds(sid * ROWS_PER_TILE, ROWS_PER_TILE)])


@functools.cache
def _edge_agg():
    return pl.kernel(
        _edge_agg_kernel,
        mesh=plsc.VectorSubcoreMesh(core_axis_name="c", subcore_axis_name="s"),
        out_type=jax.ShapeDtypeStruct((NC, NPAD, D), jnp.float32),
        scratch_types=[
            pltpu.VMEM((U, CHUNK), jnp.int32),
            pltpu.VMEM((U, CHUNK), jnp.int32),
            pltpu.VMEM((CHUNK, D), jnp.float32),
            pltpu.VMEM((CHUNK, D), jnp.float32),
            pltpu.VMEM((CHUNK, D), jnp.float32),
            pltpu.VMEM((CHUNK, D), jnp.float32),
            pltpu.VMEM((CHUNK, D), jnp.float32),
            pltpu.VMEM_SHARED((NPAD, D), jnp.float32),
            pltpu.SemaphoreType.DMA((U,)),
            pltpu.SemaphoreType.DMA((R,)),
            pltpu.SemaphoreType.DMA((R,)),
        ],
    )


N_BLK = 1000
N_GRID = N // N_BLK


def _node_kernel(x_ref, p0_ref, p1_ref, b_ref,
                 wgs_ref, wge_ref, ws1_ref, bs1_ref, ws2_ref, bs2_ref,
                 cb_ref, wc1_ref, bc1_ref, wc2_ref, bc2_ref, gam_ref, bet_ref,
                 c_sum_ref, s_sum_ref, aux_ref,
                 logit_ref, cg_ref, sg_ref, scal_ref):
    i = pl.program_id(0)
    f32 = jnp.float32
    y = x_ref[...] + p0_ref[...] + p1_ref[...]
    h_sep = jnp.maximum(jnp.dot(y, wgs_ref[...], preferred_element_type=f32), 0.0)
    s_hid = jnp.maximum(jnp.dot(h_sep, ws1_ref[...], preferred_element_type=f32)
                        + bs1_ref[...], 0.0)
    score = jax.nn.sigmoid(jnp.dot(s_hid, ws2_ref[...], preferred_element_type=f32)
                           + bs2_ref[...])
    pos = jnp.mean(score, axis=1, keepdims=True)
    nf = jnp.maximum(jnp.dot(y, wge_ref[...], preferred_element_type=f32), 0.0)
    cb = cb_ref[...]
    d2 = (jnp.sum(nf * nf, axis=1, keepdims=True)
          - 2.0 * jnp.dot(nf, cb.T, preferred_element_type=f32)
          + jnp.sum(cb * cb, axis=1)[None, :])
    m = jnp.min(d2, axis=1, keepdims=True)
    kio = lax.broadcasted_iota(jnp.int32, (N_BLK, K), 1)
    idx = jnp.min(jnp.where(d2 == m, kio, K), axis=1)
    onehot = (kio == idx[:, None]).astype(f32)
    quant = jnp.dot(onehot, cb, preferred_element_type=f32)
    res = nf + quant
    c_node = res * score
    s_node = res * (1.0 - score)
    cmt_row = jnp.sum((nf - quant) ** 2, axis=1, keepdims=True)
    b = b_ref[0]
    giota = lax.broadcasted_iota(jnp.int32, (G, N_BLK), 0)
    seg = (giota == b).astype(f32)
    ones = jnp.ones((N_BLK, 1), f32)
    zeros4 = jnp.zeros((N_BLK, 4), f32)
    auxv = jnp.concatenate([pos, 1.0 - pos, ones, cmt_row, zeros4], axis=1)
    c_part = jnp.dot(seg, c_node, preferred_element_type=f32)
    s_part = jnp.dot(seg, s_node, preferred_element_type=f32)
    a_part = jnp.dot(seg, auxv, preferred_element_type=f32)

    @pl.when(i == 0)
    def _():
        c_sum_ref[...] = c_part
        s_sum_ref[...] = s_part
        aux_ref[...] = a_part

    @pl.when(i > 0)
    def _():
        c_sum_ref[...] += c_part
        s_sum_ref[...] += s_part
        aux_ref[...] += a_part

    @pl.when(i == N_GRID - 1)
    def _():
        aux = aux_ref[...]
        cnt = jnp.maximum(aux[:, 2:3], 1.0)
        c_graph = c_sum_ref[...] / cnt
        s_graph = s_sum_ref[...] / cnt
        cg_ref[...] = c_graph
        sg_ref[...] = s_graph
        h = (jnp.dot(c_graph, wc1_ref[...], preferred_element_type=f32)
             + bc1_ref[...])
        mu = jnp.mean(h, axis=0, keepdims=True)
        var = jnp.mean((h - mu) ** 2, axis=0, keepdims=True)
        h = (h - mu) / jnp.sqrt(var + 1e-5) * gam_ref[...] + bet_ref[...]
        h = jnp.maximum(h, 0.0)
        logit_ref[...] = (jnp.dot(h, wc2_ref[...], preferred_element_type=f32)
                          + bc2_ref[...])
        pos_s = aux[:, 0:1] + 1e-8
        neg_s = aux[:, 1:2] + 1e-8
        loss_reg = jnp.mean(jnp.abs(pos_s / (pos_s + neg_s) - GAMMA))
        cmt_loss = COMMIT * jnp.sum(aux[:, 3:4]) / (N * D)
        scal_ref[...] = jnp.concatenate(
            [jnp.full((1, 1), cmt_loss, f32), jnp.full((1, 1), loss_reg, f32)],
            axis=1)


def kernel(x, edge_index, batch, Wg_sep, Wg_enc, Wsep1, bsep1, Wsep2, bsep2,
           codebook, Wc1, bc1, Wc2, bc2, bn_gamma, bn_beta):
    zeros = jnp.zeros((NPAD, D), jnp.float32)
    parts = _edge_agg()(x, edge_index[0], edge_index[1], zeros)

    batch3 = batch.reshape(N_GRID, 1, N_BLK)
    full = lambda *s: pl.BlockSpec(s, lambda i: (0,) * len(s))
    outs = pl.pallas_call(
        _node_kernel,
        grid=(N_GRID,),
        in_specs=[
            pl.BlockSpec((N_BLK, D), lambda i: (i, 0)),
            pl.BlockSpec((N_BLK, D), lambda i: (i, 0)),
            pl.BlockSpec((N_BLK, D), lambda i: (i, 0)),
            pl.BlockSpec((1, 1, N_BLK), lambda i: (i, 0, 0)),
            full(D, D), full(D, D), full(D, 2 * D), full(1, 2 * D),
            full(2 * D, D), full(1, D), full(K, D),
            full(D, 2 * D), full(1, 2 * D), full(2 * D, 1), full(1, 1),
            full(1, 2 * D), full(1, 2 * D),
        ],
        out_specs=[
            pl.BlockSpec((G, D), lambda i: (0, 0)),
            pl.BlockSpec((G, D), lambda i: (0, 0)),
            pl.BlockSpec((G, 8), lambda i: (0, 0)),
            pl.BlockSpec((G, 1), lambda i: (0, 0)),
            pl.BlockSpec((G, D), lambda i: (0, 0)),
            pl.BlockSpec((G, D), lambda i: (0, 0)),
            pl.BlockSpec((1, 2), lambda i: (0, 0)),
        ],
        out_shape=[
            jax.ShapeDtypeStruct((G, D), jnp.float32),
            jax.ShapeDtypeStruct((G, D), jnp.float32),
            jax.ShapeDtypeStruct((G, 8), jnp.float32),
            jax.ShapeDtypeStruct((G, 1), jnp.float32),
            jax.ShapeDtypeStruct((G, D), jnp.float32),
            jax.ShapeDtypeStruct((G, D), jnp.float32),
            jax.ShapeDtypeStruct((1, 2), jnp.float32),
        ],
    )(x, parts[0], parts[1], batch3,
      Wg_sep, Wg_enc, Wsep1, bsep1.reshape(1, 2 * D), Wsep2,
      bsep2.reshape(1, D), codebook,
      Wc1, bc1.reshape(1, 2 * D), Wc2, bc2.reshape(1, 1),
      bn_gamma.reshape(1, 2 * D), bn_beta.reshape(1, 2 * D))
    c_logit, c_graph, s_graph, scal = outs[3], outs[4], outs[5], outs[6]
    return (c_logit, c_graph, s_graph, scal[0, 0], scal[0, 1])
